# baseline (speedup 1.0000x reference)
import jax
import jax.numpy as jnp
from jax.experimental import pallas as pl
from jax.experimental.pallas import tpu as pltpu

_G, _N, _F = 8, 512, 128
_H = 128
_D_TP = 64
_N_MLP = 256
_N_GCN = 4
_B = 64


def _softmax(z):
    z = z - jnp.max(z, axis=-1, keepdims=True)
    e = jnp.exp(z)
    return e / jnp.sum(e, axis=-1, keepdims=True)


def _body(idx_ref, mask_ref, tp_ref, gx_ref, ga_ref,
          W_in_ref, b_in_ref, W_gcn_ref, b_gcn_ref,
          W_tp1_ref, b_tp1_ref, W_tp2_ref, b_tp2_ref,
          W_f1_ref, b_f1_ref, W_f2_ref, b_f2_ref,
          out_ref, cov_ref):
    g = pl.program_id(0)
    gx = gx_ref[0]
    ga = ga_ref[0].astype(jnp.bfloat16)

    def bdot(a, b):
        return jnp.dot(a.astype(jnp.bfloat16), b.astype(jnp.bfloat16),
                       preferred_element_type=jnp.float32)

    x = bdot(gx, W_in_ref[...])
    x = jnp.maximum(x + b_in_ref[...], 0.0)
    to_add = x
    for i in range(_N_GCN):
        z = jnp.dot(ga, x.astype(jnp.bfloat16),
                    preferred_element_type=jnp.float32)
        z = bdot(z, W_gcn_ref[i])
        z = z + b_gcn_ref[i]
        if i < _N_GCN - 1:
            x = jnp.maximum(z, 0.0)
        else:
            x = _softmax(z)
    xf = x + to_add

    m = mask_ref[...]
    denom = jnp.maximum(jnp.sum(m, axis=1, keepdims=True), 1.0)
    sel = (idx_ref[...] == g).astype(jnp.float32)
    w = m * (sel / denom)
    part = jnp.dot(w, xf, preferred_element_type=jnp.float32)

    @pl.when(g == 0)
    def _():
        cov_ref[...] = part

    @pl.when(g > 0)
    def _():
        cov_ref[...] = cov_ref[...] + part

    @pl.when(g == _G - 1)
    def _():
        cov = cov_ref[...]
        t = jnp.dot(tp_ref[...], W_tp1_ref[...],
                    preferred_element_type=jnp.float32) + b_tp1_ref[...]
        t = jnp.maximum(t, 0.0)
        t = jnp.dot(t, W_tp2_ref[...],
                    preferred_element_type=jnp.float32) + b_tp2_ref[...]
        tp_e = _softmax(t)
        h = (jnp.dot(cov, W_f1_ref[:_H],
                     preferred_element_type=jnp.float32)
             + jnp.dot(tp_e, W_f1_ref[_H:],
                       preferred_element_type=jnp.float32)
             + b_f1_ref[...])
        h = jnp.maximum(h, 0.0)
        o = jnp.dot(h, W_f2_ref[...],
                    preferred_element_type=jnp.float32) + b_f2_ref[...]
        out_ref[...] = 1.0 / (1.0 + jnp.exp(-o))


def kernel(test_parameters, graph, coverpoint_mask, graph_xs_all, graph_as_all,
           W_in, b_in, W_gcn, b_gcn, W_tp1, b_tp1, W_tp2, b_tp2,
           W_f1, b_f1, W_f2, b_f2):
    idx = graph.astype(jnp.int32)
    mask_f = coverpoint_mask.astype(jnp.float32)

    full = lambda shape: pl.BlockSpec(shape, lambda g: (0,) * len(shape))
    out = pl.pallas_call(
        _body,
        grid=(_G,),
        in_specs=[
            full((_B, 1)),
            full((_B, _N)),
            full((_B, _D_TP)),
            pl.BlockSpec((1, _N, _F), lambda g: (g, 0, 0)),
            pl.BlockSpec((1, _N, _N), lambda g: (g, 0, 0)),
            full((_F, _H)), full((_H,)),
            full((_N_GCN, _H, _H)), full((_N_GCN, _H)),
            full((_D_TP, _N_MLP)), full((_N_MLP,)),
            full((_N_MLP, _N_MLP)), full((_N_MLP,)),
            full((_H + _N_MLP, _N_MLP)), full((_N_MLP,)),
            full((_N_MLP, 1)), full((1,)),
        ],
        out_specs=pl.BlockSpec((_B, 1), lambda g: (0, 0)),
        out_shape=jax.ShapeDtypeStruct((_B, 1), jnp.float32),
        scratch_shapes=[pltpu.VMEM((_B, _H), jnp.float32)],
    )(idx, mask_f, test_parameters, graph_xs_all, graph_as_all,
      W_in, b_in, W_gcn, b_gcn, W_tp1, b_tp1, W_tp2, b_tp2,
      W_f1, b_f1, W_f2, b_f2)
    return out


# speedup vs baseline: 1.0492x; 1.0492x over previous
import jax
import jax.numpy as jnp
from jax.experimental import pallas as pl
from jax.experimental.pallas import tpu as pltpu

_G, _N, _F = 8, 512, 128
_H = 128
_D_TP = 64
_N_MLP = 256
_N_GCN = 4
_B = 64
_GPB = 2
_STEPS = _G // _GPB


def _softmax(z):
    z = z - jnp.max(z, axis=-1, keepdims=True)
    e = jnp.exp(z)
    return e / jnp.sum(e, axis=-1, keepdims=True)


def _body(idx_ref, mask_ref, tp_ref, gx_ref, ga_ref,
          W_in_ref, b_in_ref, W_gcn_ref, b_gcn_ref,
          W_tp1_ref, b_tp1_ref, W_tp2_ref, b_tp2_ref,
          W_f1_ref, b_f1_ref, W_f2_ref, b_f2_ref,
          out_ref, cov_ref):
    step = pl.program_id(0)

    def bdot(a, b):
        return jnp.dot(a.astype(jnp.bfloat16), b.astype(jnp.bfloat16),
                       preferred_element_type=jnp.float32)

    m = mask_ref[...]
    denom = jnp.maximum(jnp.sum(m, axis=1, keepdims=True), 1.0)
    md = m / denom

    part = None
    for j in range(_GPB):
        g = step * _GPB + j
        gx = gx_ref[j]
        ga = ga_ref[j].astype(jnp.bfloat16)
        x = bdot(gx, W_in_ref[...])
        x = jnp.maximum(x + b_in_ref[...], 0.0)
        to_add = x
        for i in range(_N_GCN):
            z = jnp.dot(ga, x.astype(jnp.bfloat16),
                        preferred_element_type=jnp.float32)
            z = bdot(z, W_gcn_ref[i])
            z = z + b_gcn_ref[i]
            if i < _N_GCN - 1:
                x = jnp.maximum(z, 0.0)
            else:
                x = _softmax(z)
        xf = x + to_add

        sel = (idx_ref[...] == g).astype(jnp.float32)
        w = md * sel
        p = jnp.dot(w, xf, preferred_element_type=jnp.float32)
        part = p if part is None else part + p

    @pl.when(step == 0)
    def _():
        cov_ref[...] = part

    @pl.when(step > 0)
    def _():
        cov_ref[...] = cov_ref[...] + part

    @pl.when(step == _STEPS - 1)
    def _():
        cov = cov_ref[...]
        t = jnp.dot(tp_ref[...], W_tp1_ref[...],
                    preferred_element_type=jnp.float32) + b_tp1_ref[...]
        t = jnp.maximum(t, 0.0)
        t = jnp.dot(t, W_tp2_ref[...],
                    preferred_element_type=jnp.float32) + b_tp2_ref[...]
        tp_e = _softmax(t)
        h = (jnp.dot(cov, W_f1_ref[:_H],
                     preferred_element_type=jnp.float32)
             + jnp.dot(tp_e, W_f1_ref[_H:],
                       preferred_element_type=jnp.float32)
             + b_f1_ref[...])
        h = jnp.maximum(h, 0.0)
        o = jnp.dot(h, W_f2_ref[...],
                    preferred_element_type=jnp.float32) + b_f2_ref[...]
        out_ref[...] = 1.0 / (1.0 + jnp.exp(-o))


def kernel(test_parameters, graph, coverpoint_mask, graph_xs_all, graph_as_all,
           W_in, b_in, W_gcn, b_gcn, W_tp1, b_tp1, W_tp2, b_tp2,
           W_f1, b_f1, W_f2, b_f2):
    idx = graph.astype(jnp.int32)
    mask_f = coverpoint_mask.astype(jnp.float32)

    full = lambda shape: pl.BlockSpec(shape, lambda g: (0,) * len(shape))
    out = pl.pallas_call(
        _body,
        grid=(_STEPS,),
        in_specs=[
            full((_B, 1)),
            full((_B, _N)),
            full((_B, _D_TP)),
            pl.BlockSpec((_GPB, _N, _F), lambda g: (g, 0, 0)),
            pl.BlockSpec((_GPB, _N, _N), lambda g: (g, 0, 0)),
            full((_F, _H)), full((_H,)),
            full((_N_GCN, _H, _H)), full((_N_GCN, _H)),
            full((_D_TP, _N_MLP)), full((_N_MLP,)),
            full((_N_MLP, _N_MLP)), full((_N_MLP,)),
            full((_H + _N_MLP, _N_MLP)), full((_N_MLP,)),
            full((_N_MLP, 1)), full((1,)),
        ],
        out_specs=pl.BlockSpec((_B, 1), lambda g: (0, 0)),
        out_shape=jax.ShapeDtypeStruct((_B, 1), jnp.float32),
        scratch_shapes=[pltpu.VMEM((_B, _H), jnp.float32)],
    )(idx, mask_f, test_parameters, graph_xs_all, graph_as_all,
      W_in, b_in, W_gcn, b_gcn, W_tp1, b_tp1, W_tp2, b_tp2,
      W_f1, b_f1, W_f2, b_f2)
    return out
